# baseline (speedup 1.0000x reference)
import functools
import math

import jax
import jax.numpy as jnp
from jax import lax
from jax.experimental import pallas as pl
from jax.experimental.pallas import tpu as pltpu
from jax.experimental.pallas import tpu_sc as plsc

_NC = 2
_NS = 16
_NW = _NC * _NS

_IDX_W = 128


def _gather_sc(ids_flat, lut, sub_chunks):
    n_tok = ids_flat.shape[0]
    d = lut.shape[1]
    per_w = n_tok // _NW
    chunk = sub_chunks * _IDX_W
    n_chunks = per_w // chunk
    assert n_chunks * chunk == per_w
    ids2 = ids_flat.reshape(n_tok // _IDX_W, _IDX_W)

    mesh = plsc.VectorSubcoreMesh(core_axis_name="c", subcore_axis_name="s")

    @functools.partial(
        pl.kernel,
        mesh=mesh,
        out_type=jax.ShapeDtypeStruct((n_tok, 2 * d), jnp.float32),
        scratch_types=[
            pltpu.VMEM((sub_chunks, _IDX_W), jnp.int32),
            pltpu.VMEM((chunk, d), jnp.float32),
            pltpu.SemaphoreType.DMA,
        ],
        compiler_params=pltpu.CompilerParams(use_tc_tiling_on_sc=False),
    )
    def gather_kernel(ids_hbm, lut_hbm, out_hbm, idx_v, rows_v, sem):
        wid = lax.axis_index("s") * _NC + lax.axis_index("c")
        base = wid * per_w
        base_row = wid * (per_w // _IDX_W)

        def body(j, carry):
            off = base + j * chunk
            pltpu.sync_copy(
                ids_hbm.at[pl.ds(base_row + j * sub_chunks, sub_chunks)],
                idx_v,
            )
            copies = [
                pltpu.async_copy(
                    lut_hbm.at[idx_v.at[k]],
                    rows_v.at[pl.ds(k * _IDX_W, _IDX_W)],
                    sem,
                )
                for k in range(sub_chunks)
            ]
            for cp in copies:
                cp.wait()
            pltpu.sync_copy(rows_v, out_hbm.at[pl.ds(off, chunk), pl.ds(0, d)])
            return carry

        lax.fori_loop(0, n_chunks, body, 0)

    return gather_kernel(ids2, lut)


def _pack_lut(lut, v_block):
    d, v = lut.shape[1], lut.shape[0]
    lut_t = jnp.transpose(lut, (1, 0))
    grid = (v + v_block - 1) // v_block

    def pack_kernel(t_ref, o_ref):
        r = lax.broadcasted_iota(jnp.int32, (d, d), 0)
        c = lax.broadcasted_iota(jnp.int32, (d, d), 1)
        eye = (r == c).astype(jnp.float32)
        tt = jax.lax.dot_general(
            t_ref[...], eye, (((0,), (0,)), ((), ())),
            preferred_element_type=jnp.float32,
        )
        ttr = tt.reshape(v_block // 2, 2, d)
        o_ref[...] = jnp.concatenate([ttr[:, 0, :], ttr[:, 1, :]], axis=1)

    packed = pl.pallas_call(
        pack_kernel,
        grid=(grid,),
        in_specs=[pl.BlockSpec((d, v_block), lambda i: (0, i))],
        out_specs=pl.BlockSpec((v_block // 2, 2 * d), lambda i: (i, 0)),
        out_shape=jax.ShapeDtypeStruct((v // 2, 2 * d), jnp.float32),
    )(lut_t)
    return packed.reshape(v, d)


def _sent_tc(sent_repr, w_merge, b_col):
    batch, d = sent_repr.shape
    scale = math.sqrt(d)

    def sent_kernel(s_ref, w_ref, b_ref, o_ref):
        wt = w_ref[...] * scale
        y = jax.lax.dot_general(
            s_ref[...], wt, (((1,), (0,)), ((), ())),
            preferred_element_type=jnp.float32,
        )
        o_ref[...] = jnp.transpose(y, (1, 0)) + b_ref[...]

    return pl.pallas_call(
        sent_kernel,
        in_specs=[
            pl.BlockSpec((batch, d), lambda: (0, 0)),
            pl.BlockSpec((d, d), lambda: (0, 0)),
            pl.BlockSpec((d, 1), lambda: (0, 0)),
        ],
        out_specs=pl.BlockSpec((d, batch), lambda: (0, 0)),
        out_shape=jax.ShapeDtypeStruct((d, batch), jnp.float32),
    )(sent_repr, w_merge, b_col)


def _merge_tc(gathered_w, smT, w_merge, batch, seq, l_block):
    n_tok, two_d = gathered_w.shape
    d = two_d // 2
    grid = seq // l_block
    scale = math.sqrt(d)

    def merge_kernel(g_ref, s_ref, w_ref, o_ref):
        wb = w_ref[...] * scale
        smt = s_ref[...]
        for j in range(l_block):
            g = g_ref[pl.ds(j * batch, batch), :][:, :d]
            y = jax.lax.dot_general(
                g, wb, (((1,), (0,)), ((), ())),
                preferred_element_type=jnp.float32,
            )
            o_ref[pl.ds(j * d, d), :] = jnp.transpose(y, (1, 0)) + smt

    return pl.pallas_call(
        merge_kernel,
        grid=(grid,),
        in_specs=[
            pl.BlockSpec((l_block * batch, two_d), lambda i: (i, 0)),
            pl.BlockSpec((d, batch), lambda i: (0, 0)),
            pl.BlockSpec((d, d), lambda i: (0, 0)),
        ],
        out_specs=pl.BlockSpec((l_block * d, batch), lambda i: (i, 0)),
        out_shape=jax.ShapeDtypeStruct((seq * d, batch), jnp.float32),
    )(gathered_w, smT, w_merge[d:, :])


def kernel(word_ids, sent_repr, lut, W_merge, b_merge):
    batch, seq = word_ids.shape
    d = lut.shape[1]
    ids_t = jnp.transpose(word_ids, (1, 0)).reshape(-1).astype(jnp.int32)
    table = _pack_lut(lut, v_block=2048)
    gathered_w = _gather_sc(ids_t, table, sub_chunks=8)
    smT = _sent_tc(sent_repr, W_merge[:d, :], b_merge.reshape(d, 1))
    out2d = _merge_tc(gathered_w, smT, W_merge, batch, seq, l_block=4)
    return jnp.transpose(out2d.reshape(seq, d, batch), (2, 0, 1))


# speedup vs baseline: 1.0315x; 1.0315x over previous
import functools
import math

import jax
import jax.numpy as jnp
from jax import lax
from jax.experimental import pallas as pl
from jax.experimental.pallas import tpu as pltpu
from jax.experimental.pallas import tpu_sc as plsc

_NC = 2
_NS = 16
_NW = _NC * _NS

_IDX_W = 128


def _gather_sc(ids_flat, lut, sub_chunks):
    n_tok = ids_flat.shape[0]
    d = lut.shape[1]
    per_w = n_tok // _NW
    chunk = sub_chunks * _IDX_W
    n_chunks = per_w // chunk
    assert n_chunks * chunk == per_w
    ids2 = ids_flat.reshape(n_tok // _IDX_W, _IDX_W)

    mesh = plsc.VectorSubcoreMesh(core_axis_name="c", subcore_axis_name="s")

    @functools.partial(
        pl.kernel,
        mesh=mesh,
        out_type=jax.ShapeDtypeStruct((n_tok, 2 * d), jnp.float32),
        scratch_types=[
            pltpu.VMEM((sub_chunks, _IDX_W), jnp.int32),
            pltpu.VMEM((chunk, d), jnp.float32),
            pltpu.SemaphoreType.DMA,
        ],
        compiler_params=pltpu.CompilerParams(use_tc_tiling_on_sc=False),
    )
    def gather_kernel(ids_hbm, lut_hbm, out_hbm, idx_v, rows_v, sem):
        wid = lax.axis_index("s") * _NC + lax.axis_index("c")
        base = wid * per_w
        base_row = wid * (per_w // _IDX_W)

        def body(j, carry):
            off = base + j * chunk
            pltpu.sync_copy(
                ids_hbm.at[pl.ds(base_row + j * sub_chunks, sub_chunks)],
                idx_v,
            )
            copies = [
                pltpu.async_copy(
                    lut_hbm.at[idx_v.at[k]],
                    rows_v.at[pl.ds(k * _IDX_W, _IDX_W)],
                    sem,
                )
                for k in range(sub_chunks)
            ]
            for cp in copies:
                cp.wait()
            pltpu.sync_copy(rows_v, out_hbm.at[pl.ds(off, chunk), pl.ds(0, d)])
            return carry

        lax.fori_loop(0, n_chunks, body, 0)

    return gather_kernel(ids2, lut)


def _pack_lut(lut, v_block):
    d, v = lut.shape[1], lut.shape[0]
    lut_t = jnp.transpose(lut, (1, 0))
    grid = (v + v_block - 1) // v_block

    def pack_kernel(t_ref, o_ref):
        tt = jnp.transpose(t_ref[...], (1, 0))
        ttr = tt.reshape(v_block // 2, 2, d)
        o_ref[...] = jnp.concatenate([ttr[:, 0, :], ttr[:, 1, :]], axis=1)

    packed = pl.pallas_call(
        pack_kernel,
        grid=(grid,),
        in_specs=[pl.BlockSpec((d, v_block), lambda i: (0, i))],
        out_specs=pl.BlockSpec((v_block // 2, 2 * d), lambda i: (i, 0)),
        out_shape=jax.ShapeDtypeStruct((v // 2, 2 * d), jnp.float32),
    )(lut_t)
    return packed.reshape(v, d)


def _sent_tc(sent_repr, w_merge, b_col):
    batch, d = sent_repr.shape
    scale = math.sqrt(d)

    def sent_kernel(s_ref, w_ref, b_ref, o_ref):
        wt = w_ref[...] * scale
        y = jax.lax.dot_general(
            s_ref[...], wt, (((1,), (0,)), ((), ())),
            preferred_element_type=jnp.float32,
        )
        o_ref[...] = jnp.transpose(y, (1, 0)) + b_ref[...]

    return pl.pallas_call(
        sent_kernel,
        in_specs=[
            pl.BlockSpec((batch, d), lambda: (0, 0)),
            pl.BlockSpec((d, d), lambda: (0, 0)),
            pl.BlockSpec((d, 1), lambda: (0, 0)),
        ],
        out_specs=pl.BlockSpec((d, batch), lambda: (0, 0)),
        out_shape=jax.ShapeDtypeStruct((d, batch), jnp.float32),
    )(sent_repr, w_merge, b_col)


def _merge_tc(gathered_w, smT, w_merge, batch, total_seq, l_off, l_block,
              carry=None):
    n_tok, two_d = gathered_w.shape
    d = two_d // 2
    seq_h = n_tok // batch
    grid = seq_h // l_block
    off_blk = l_off // l_block
    scale = math.sqrt(d)

    def merge_kernel(g_ref, s_ref, w_ref, *rest):
        o_ref = rest[-1]
        wb = w_ref[...] * scale
        smt = s_ref[...]
        for j in range(l_block):
            g = g_ref[pl.ds(j * batch, batch), :][:, :d]
            y = jax.lax.dot_general(
                g, wb, (((1,), (0,)), ((), ())),
                preferred_element_type=jnp.float32,
            )
            o_ref[pl.ds(j * d, d), :] = jnp.transpose(y, (1, 0)) + smt

    in_specs = [
        pl.BlockSpec((l_block * batch, two_d), lambda i: (i, 0)),
        pl.BlockSpec((d, batch), lambda i: (0, 0)),
        pl.BlockSpec((d, d), lambda i: (0, 0)),
    ]
    args = [gathered_w, smT, w_merge[d:, :]]
    aliases = {}
    if carry is not None:
        in_specs.append(pl.BlockSpec(memory_space=pl.ANY))
        args.append(carry)
        aliases = {3: 0}

    return pl.pallas_call(
        merge_kernel,
        grid=(grid,),
        in_specs=in_specs,
        out_specs=pl.BlockSpec(
            (l_block * d, batch), lambda i, off_blk=off_blk: (i + off_blk, 0)
        ),
        out_shape=jax.ShapeDtypeStruct((total_seq * d, batch), jnp.float32),
        input_output_aliases=aliases,
    )(*args)


def kernel(word_ids, sent_repr, lut, W_merge, b_merge):
    batch, seq = word_ids.shape
    d = lut.shape[1]
    ids_t = jnp.transpose(word_ids, (1, 0)).reshape(-1).astype(jnp.int32)
    table = _pack_lut(lut, v_block=2048)
    smT = _sent_tc(sent_repr, W_merge[:d, :], b_merge.reshape(d, 1))
    half = seq // 2
    n_half = batch * half
    g1 = _gather_sc(ids_t[:n_half], table, sub_chunks=4)
    g2 = _gather_sc(ids_t[n_half:], table, sub_chunks=4)
    m1 = _merge_tc(g1, smT, W_merge, batch, seq, l_off=0, l_block=4)
    out2d = _merge_tc(
        g2, smT, W_merge, batch, seq, l_off=half, l_block=4, carry=m1
    )
    return jnp.transpose(out2d.reshape(seq, d, batch), (2, 0, 1))


# speedup vs baseline: 1.1810x; 1.1450x over previous
import functools
import math

import jax
import jax.numpy as jnp
from jax import lax
from jax.experimental import pallas as pl
from jax.experimental.pallas import tpu as pltpu
from jax.experimental.pallas import tpu_sc as plsc

_NC = 2
_NS = 16
_NW = _NC * _NS

_IDX_W = 128


def _gather_sc(ids_flat, lut, sub_chunks):
    n_tok = ids_flat.shape[0]
    d = lut.shape[1]
    per_w = n_tok // _NW
    chunk = sub_chunks * _IDX_W
    n_chunks = per_w // chunk
    assert n_chunks * chunk == per_w
    ids2 = ids_flat.reshape(n_tok // _IDX_W, _IDX_W)

    mesh = plsc.VectorSubcoreMesh(core_axis_name="c", subcore_axis_name="s")

    @functools.partial(
        pl.kernel,
        mesh=mesh,
        out_type=jax.ShapeDtypeStruct((n_tok, 2 * d), jnp.float32),
        scratch_types=[
            pltpu.VMEM((sub_chunks, _IDX_W), jnp.int32),
            pltpu.VMEM((chunk, d), jnp.float32),
            pltpu.SemaphoreType.DMA,
        ],
        compiler_params=pltpu.CompilerParams(use_tc_tiling_on_sc=False),
    )
    def gather_kernel(ids_hbm, lut_hbm, out_hbm, idx_v, rows_v, sem):
        wid = lax.axis_index("s") * _NC + lax.axis_index("c")
        base = wid * per_w
        base_row = wid * (per_w // _IDX_W)

        def body(j, carry):
            off = base + j * chunk
            pltpu.sync_copy(
                ids_hbm.at[pl.ds(base_row + j * sub_chunks, sub_chunks)],
                idx_v,
            )
            copies = [
                pltpu.async_copy(
                    lut_hbm.at[idx_v.at[k]],
                    rows_v.at[pl.ds(k * _IDX_W, _IDX_W)],
                    sem,
                )
                for k in range(sub_chunks)
            ]
            for cp in copies:
                cp.wait()
            pltpu.sync_copy(rows_v, out_hbm.at[pl.ds(off, chunk), pl.ds(0, d)])
            return carry

        lax.fori_loop(0, n_chunks, body, 0)

    return gather_kernel(ids2, lut)


def _pack_lut(lut, v_block):
    d, v = lut.shape[1], lut.shape[0]
    lut_t = jnp.transpose(lut, (1, 0))
    grid = (v + v_block - 1) // v_block

    def pack_kernel(t_ref, o_ref):
        tt = jnp.transpose(t_ref[...], (1, 0))
        ttr = tt.reshape(v_block // 2, 2, d)
        o_ref[...] = jnp.concatenate([ttr[:, 0, :], ttr[:, 1, :]], axis=1)

    packed = pl.pallas_call(
        pack_kernel,
        grid=(grid,),
        in_specs=[pl.BlockSpec((d, v_block), lambda i: (0, i))],
        out_specs=pl.BlockSpec((v_block // 2, 2 * d), lambda i: (i, 0)),
        out_shape=jax.ShapeDtypeStruct((v // 2, 2 * d), jnp.float32),
    )(lut_t)
    return packed.reshape(v, d)


def _sent_tc(sent_repr, w_merge, b_col):
    batch, d = sent_repr.shape
    scale = math.sqrt(d)

    def sent_kernel(s_ref, w_ref, b_ref, o_ref):
        wt = w_ref[...] * scale
        y = jax.lax.dot_general(
            s_ref[...], wt, (((1,), (0,)), ((), ())),
            preferred_element_type=jnp.float32,
        )
        o_ref[...] = jnp.transpose(y, (1, 0)) + b_ref[...]

    return pl.pallas_call(
        sent_kernel,
        in_specs=[
            pl.BlockSpec((batch, d), lambda: (0, 0)),
            pl.BlockSpec((d, d), lambda: (0, 0)),
            pl.BlockSpec((d, 1), lambda: (0, 0)),
        ],
        out_specs=pl.BlockSpec((d, batch), lambda: (0, 0)),
        out_shape=jax.ShapeDtypeStruct((d, batch), jnp.float32),
    )(sent_repr, w_merge, b_col)


def _merge_tc(gathered_w, smT, w_merge, batch, total_seq, l_off, l_block,
              carry=None):
    n_tok, two_d = gathered_w.shape
    d = two_d // 2
    seq_h = n_tok // batch
    grid = seq_h // l_block
    off_blk = l_off // l_block
    scale = math.sqrt(d)

    def merge_kernel(g_ref, s_ref, w_ref, *rest):
        o_ref = rest[-1]
        wb = w_ref[...] * scale
        smt = s_ref[...]
        for j in range(l_block):
            g = g_ref[pl.ds(j * batch, batch), :][:, :d]
            y = jax.lax.dot_general(
                g, wb, (((1,), (0,)), ((), ())),
                preferred_element_type=jnp.float32,
            )
            o_ref[pl.ds(j * d, d), :] = jnp.transpose(y, (1, 0)) + smt

    in_specs = [
        pl.BlockSpec((l_block * batch, two_d), lambda i: (i, 0)),
        pl.BlockSpec((d, batch), lambda i: (0, 0)),
        pl.BlockSpec((d, d), lambda i: (0, 0)),
    ]
    args = [gathered_w, smT, w_merge[d:, :]]
    aliases = {}
    if carry is not None:
        in_specs.append(pl.BlockSpec(memory_space=pl.ANY))
        args.append(carry)
        aliases = {3: 0}

    return pl.pallas_call(
        merge_kernel,
        grid=(grid,),
        in_specs=in_specs,
        out_specs=pl.BlockSpec(
            (l_block * d, batch), lambda i, off_blk=off_blk: (i + off_blk, 0)
        ),
        out_shape=jax.ShapeDtypeStruct((total_seq * d, batch), jnp.float32),
        input_output_aliases=aliases,
    )(*args)


def kernel(word_ids, sent_repr, lut, W_merge, b_merge):
    batch, seq = word_ids.shape
    d = lut.shape[1]
    ids_t = jnp.transpose(word_ids, (1, 0)).reshape(-1).astype(jnp.int32)
    table = _pack_lut(lut, v_block=4096)
    smT = _sent_tc(sent_repr, W_merge[:d, :], b_merge.reshape(d, 1))
    gathered_w = _gather_sc(ids_t, table, sub_chunks=8)
    out2d = _merge_tc(gathered_w, smT, W_merge, batch, seq, l_off=0, l_block=4)
    return jnp.transpose(out2d.reshape(seq, d, batch), (2, 0, 1))


# speedup vs baseline: 1.2163x; 1.0298x over previous
import functools
import math

import jax
import jax.numpy as jnp
from jax import lax
from jax.experimental import pallas as pl
from jax.experimental.pallas import tpu as pltpu
from jax.experimental.pallas import tpu_sc as plsc

_NC = 2
_NS = 16
_NW = _NC * _NS

_IDX_W = 128


def _gather_sc(ids_flat, lut, sub_chunks):
    n_tok = ids_flat.shape[0]
    d = lut.shape[1]
    per_w = n_tok // _NW
    chunk = sub_chunks * _IDX_W
    n_chunks = per_w // chunk
    assert n_chunks * chunk == per_w
    ids2 = ids_flat.reshape(n_tok // _IDX_W, _IDX_W)

    mesh = plsc.VectorSubcoreMesh(core_axis_name="c", subcore_axis_name="s")

    @functools.partial(
        pl.kernel,
        mesh=mesh,
        out_type=jax.ShapeDtypeStruct((n_tok, 2 * d), jnp.float32),
        scratch_types=[
            pltpu.VMEM((sub_chunks, _IDX_W), jnp.int32),
            pltpu.VMEM((chunk, d), jnp.float32),
            pltpu.SemaphoreType.DMA,
        ],
        compiler_params=pltpu.CompilerParams(use_tc_tiling_on_sc=False),
    )
    def gather_kernel(ids_hbm, lut_hbm, out_hbm, idx_v, rows_v, sem):
        wid = lax.axis_index("s") * _NC + lax.axis_index("c")
        base = wid * per_w
        base_row = wid * (per_w // _IDX_W)

        def body(j, carry):
            off = base + j * chunk
            pltpu.sync_copy(
                ids_hbm.at[pl.ds(base_row + j * sub_chunks, sub_chunks)],
                idx_v,
            )
            copies = [
                pltpu.async_copy(
                    lut_hbm.at[idx_v.at[k]],
                    rows_v.at[pl.ds(k * _IDX_W, _IDX_W)],
                    sem,
                )
                for k in range(sub_chunks)
            ]
            for cp in copies:
                cp.wait()
            pltpu.sync_copy(rows_v, out_hbm.at[pl.ds(off, chunk), pl.ds(0, d)])
            return carry

        lax.fori_loop(0, n_chunks, body, 0)

    return gather_kernel(ids2, lut)


def _pack_lut(lut, v_block):
    d, v = lut.shape[1], lut.shape[0]
    lut_t = jnp.transpose(lut, (1, 0))
    grid = (v + v_block - 1) // v_block

    def pack_kernel(t_ref, o_ref):
        tt = jnp.transpose(t_ref[...], (1, 0))
        ttr = tt.reshape(v_block // 2, 2, d)
        o_ref[...] = jnp.concatenate([ttr[:, 0, :], ttr[:, 1, :]], axis=1)

    packed = pl.pallas_call(
        pack_kernel,
        grid=(grid,),
        in_specs=[pl.BlockSpec((d, v_block), lambda i: (0, i))],
        out_specs=pl.BlockSpec((v_block // 2, 2 * d), lambda i: (i, 0)),
        out_shape=jax.ShapeDtypeStruct((v // 2, 2 * d), jnp.float32),
    )(lut_t)
    return packed.reshape(v, d)


def _sent_tc(sent_repr, w_merge, b_col):
    batch, d = sent_repr.shape
    scale = math.sqrt(d)

    def sent_kernel(s_ref, w_ref, b_ref, o_ref):
        wt = w_ref[...] * scale
        y = jax.lax.dot_general(
            s_ref[...], wt, (((1,), (0,)), ((), ())),
            preferred_element_type=jnp.float32,
        )
        o_ref[...] = jnp.transpose(y, (1, 0)) + b_ref[...]

    return pl.pallas_call(
        sent_kernel,
        in_specs=[
            pl.BlockSpec((batch, d), lambda: (0, 0)),
            pl.BlockSpec((d, d), lambda: (0, 0)),
            pl.BlockSpec((d, 1), lambda: (0, 0)),
        ],
        out_specs=pl.BlockSpec((d, batch), lambda: (0, 0)),
        out_shape=jax.ShapeDtypeStruct((d, batch), jnp.float32),
    )(sent_repr, w_merge, b_col)


def _merge_tc(gathered_w, smT, w_merge, batch, total_seq, l_off, l_block,
              carry=None):
    n_tok, two_d = gathered_w.shape
    d = two_d // 2
    seq_h = n_tok // batch
    grid = seq_h // l_block
    off_blk = l_off // l_block
    scale = math.sqrt(d)

    def merge_kernel(g_ref, s_ref, w_ref, *rest):
        o_ref = rest[-1]
        wb = w_ref[...] * scale
        smt = s_ref[...]
        for j in range(l_block):
            g = g_ref[pl.ds(j * batch, batch), :][:, :d]
            y = jax.lax.dot_general(
                g, wb, (((1,), (0,)), ((), ())),
                preferred_element_type=jnp.float32,
            )
            o_ref[pl.ds(j * d, d), :] = jnp.transpose(y, (1, 0)) + smt

    in_specs = [
        pl.BlockSpec((l_block * batch, two_d), lambda i: (i, 0)),
        pl.BlockSpec((d, batch), lambda i: (0, 0)),
        pl.BlockSpec((d, d), lambda i: (0, 0)),
    ]
    args = [gathered_w, smT, w_merge[d:, :]]
    aliases = {}
    if carry is not None:
        in_specs.append(pl.BlockSpec(memory_space=pl.ANY))
        args.append(carry)
        aliases = {3: 0}

    return pl.pallas_call(
        merge_kernel,
        grid=(grid,),
        in_specs=in_specs,
        out_specs=pl.BlockSpec(
            (l_block * d, batch), lambda i, off_blk=off_blk: (i + off_blk, 0)
        ),
        out_shape=jax.ShapeDtypeStruct((total_seq * d, batch), jnp.float32),
        input_output_aliases=aliases,
    )(*args)


def kernel(word_ids, sent_repr, lut, W_merge, b_merge):
    batch, seq = word_ids.shape
    d = lut.shape[1]
    ids_t = jnp.transpose(word_ids, (1, 0)).reshape(-1).astype(jnp.int32)
    table = _pack_lut(lut, v_block=8192)
    smT = _sent_tc(sent_repr, W_merge[:d, :], b_merge.reshape(d, 1))
    gathered_w = _gather_sc(ids_t, table, sub_chunks=8)
    out2d = _merge_tc(gathered_w, smT, W_merge, batch, seq, l_off=0, l_block=4)
    return jnp.transpose(out2d.reshape(seq, d, batch), (2, 0, 1))
